# baseline (speedup 1.0000x reference)
import functools

import jax
import jax.numpy as jnp
from jax import lax
from jax.experimental import pallas as pl
from jax.experimental.pallas import tpu as pltpu
from jax.experimental.pallas import tpu_sc as plsc

N_NODES = 10000
DIM = 128
N_EDGES = 320000

NC = 2
NS = 16
CHUNK = 128
HC = CHUNK // 2
NCHUNK = 160
GSTAGE = 40
N_PAD = 10240
ROWS_PER_TILE = N_PAD // NS
ROW_CHUNKS = ROWS_PER_TILE // CHUNK


def _gather2(src_hbm, gi_v, j, rbuf, sa, sb):
    pltpu.async_copy(src_hbm.at[gi_v.at[j, pl.ds(0, HC)]],
                     rbuf.at[pl.ds(0, HC)], sa)
    pltpu.async_copy(src_hbm.at[gi_v.at[j, pl.ds(HC, HC)]],
                     rbuf.at[pl.ds(HC, HC)], sb)


def _wait2(src_hbm, gi_v, j, rbuf, sa, sb):
    pltpu.make_async_copy(src_hbm.at[gi_v.at[j, pl.ds(0, HC)]],
                          rbuf.at[pl.ds(0, HC)], sa).wait()
    pltpu.make_async_copy(src_hbm.at[gi_v.at[j, pl.ds(HC, HC)]],
                          rbuf.at[pl.ds(HC, HC)], sb).wait()


def _sc_propagate_body(x0_hbm, gidx_hbm, outs_hbm,
                       gi_v, si_v, rbuf0, rbuf1, acc, s0a, s0b, s1a, s1b):
    c = lax.axis_index("c")
    s = lax.axis_index("s")
    row0 = s * ROWS_PER_TILE

    for j in range(ROW_CHUNKS):
        r = row0 + j * CHUNK
        pltpu.sync_copy(x0_hbm.at[pl.ds(r, CHUNK)], rbuf0)
        pltpu.sync_copy(rbuf0, acc.at[pl.ds(r, CHUNK)])
    plsc.subcore_barrier()

    for hop in range(2):
        src_hbm = x0_hbm if hop == 0 else outs_hbm.at[c, 0]

        for stg in range(NCHUNK // GSTAGE):
            pltpu.sync_copy(gidx_hbm.at[c, s, pl.ds(stg * GSTAGE, GSTAGE)], gi_v)
            pltpu.sync_copy(gidx_hbm.at[1 - c, s, pl.ds(stg * GSTAGE, GSTAGE)], si_v)
            _gather2(src_hbm, gi_v, 0, rbuf0, s0a, s0b)

            def edge_pair(m, carry):
                b = 2 * m
                _wait2(src_hbm, gi_v, b, rbuf0, s0a, s0b)
                _gather2(src_hbm, gi_v, b + 1, rbuf1, s1a, s1b)
                pltpu.sync_copy(rbuf0, acc.at[si_v.at[b]], add=True)
                _wait2(src_hbm, gi_v, b + 1, rbuf1, s1a, s1b)

                @pl.when(m < GSTAGE // 2 - 1)
                def _():
                    _gather2(src_hbm, gi_v, b + 2, rbuf0, s0a, s0b)

                pltpu.sync_copy(rbuf1, acc.at[si_v.at[b + 1]], add=True)
                return carry

            lax.fori_loop(0, GSTAGE // 2, edge_pair, 0)
        plsc.subcore_barrier()

        for j in range(ROW_CHUNKS):
            r = row0 + j * CHUNK
            pltpu.sync_copy(acc.at[pl.ds(r, CHUNK)], rbuf0)
            pltpu.sync_copy(rbuf0, outs_hbm.at[c, hop, pl.ds(r, CHUNK)])
        plsc.subcore_barrier()


_sc_propagate = functools.partial(
    pl.kernel,
    out_type=jax.ShapeDtypeStruct((NC, 2, N_PAD, DIM), jnp.float32),
    mesh=plsc.VectorSubcoreMesh(core_axis_name="c", subcore_axis_name="s"),
    scratch_types=[
        pltpu.VMEM((GSTAGE, CHUNK), jnp.int32),
        pltpu.VMEM((GSTAGE, CHUNK), jnp.int32),
        pltpu.VMEM((CHUNK, DIM), jnp.float32),
        pltpu.VMEM((CHUNK, DIM), jnp.float32),
        pltpu.VMEM_SHARED((N_PAD, DIM), jnp.float32),
        pltpu.SemaphoreType.DMA,
        pltpu.SemaphoreType.DMA,
        pltpu.SemaphoreType.DMA,
        pltpu.SemaphoreType.DMA,
    ],
)(_sc_propagate_body)


def _mlp_body(x0_ref, hops_ref, W1_ref, b1_ref, W2_ref, b2_ref, o_ref):
    x0 = x0_ref[...]
    h = jnp.concatenate(
        [x0, hops_ref[0, 0], hops_ref[0, 1], x0, hops_ref[1, 0], hops_ref[1, 1]],
        axis=-1,
    )
    h = jnp.dot(h, W1_ref[...], preferred_element_type=jnp.float32) + b1_ref[...]
    h = jnp.maximum(h, 0.0)
    o_ref[...] = (
        jnp.dot(h, W2_ref[...], preferred_element_type=jnp.float32) + b2_ref[...]
    )


def _mlp(x0p, hops, W1, b1, W2, b2):
    blk = 2000
    grid = N_NODES // blk
    return pl.pallas_call(
        _mlp_body,
        grid=(grid,),
        in_specs=[
            pl.BlockSpec((blk, DIM), lambda i: (i, 0)),
            pl.BlockSpec((NC, 2, blk, DIM), lambda i: (0, 0, i, 0)),
            pl.BlockSpec((2 * 3 * DIM, 512), lambda i: (0, 0)),
            pl.BlockSpec((1, 512), lambda i: (0, 0)),
            pl.BlockSpec((512, DIM), lambda i: (0, 0)),
            pl.BlockSpec((1, DIM), lambda i: (0, 0)),
        ],
        out_specs=pl.BlockSpec((blk, DIM), lambda i: (i, 0)),
        out_shape=jax.ShapeDtypeStruct((N_NODES, DIM), jnp.float32),
    )(x0p, hops, W1, b1.reshape(1, -1), W2, b2.reshape(1, -1))


def kernel(feature, edge_index, W1, b1, W2, b2):
    pad = NS * NCHUNK * CHUNK - N_EDGES
    padv = jnp.full((2, pad), N_NODES, jnp.int32)
    gidx = jnp.concatenate([edge_index, padv], axis=1).reshape(
        2, NS, NCHUNK, CHUNK)
    x0p = jnp.concatenate(
        [feature, jnp.zeros((N_PAD - N_NODES, DIM), jnp.float32)]
    )
    hops = _sc_propagate(x0p, gidx)
    return _mlp(x0p, hops, W1, b1, W2, b2)


# speedup vs baseline: 1.0402x; 1.0402x over previous
import functools

import jax
import jax.numpy as jnp
from jax import lax
from jax.experimental import pallas as pl
from jax.experimental.pallas import tpu as pltpu
from jax.experimental.pallas import tpu_sc as plsc

N_NODES = 10000
DIM = 128
N_EDGES = 320000

NC = 2
NS = 16
CHUNK = 128
HC = CHUNK // 2
NCHUNK = 160
GSTAGE = 40
N_PAD = 10240
ROWS_PER_TILE = N_PAD // NS
ROW_CHUNKS = ROWS_PER_TILE // CHUNK


def _gather2(src_hbm, gi_v, j, rbuf, sa, sb):
    pltpu.async_copy(src_hbm.at[gi_v.at[j, pl.ds(0, HC)]],
                     rbuf.at[pl.ds(0, HC)], sa)
    pltpu.async_copy(src_hbm.at[gi_v.at[j, pl.ds(HC, HC)]],
                     rbuf.at[pl.ds(HC, HC)], sb)


def _wait2(src_hbm, gi_v, j, rbuf, sa, sb):
    pltpu.make_async_copy(src_hbm.at[gi_v.at[j, pl.ds(0, HC)]],
                          rbuf.at[pl.ds(0, HC)], sa).wait()
    pltpu.make_async_copy(src_hbm.at[gi_v.at[j, pl.ds(HC, HC)]],
                          rbuf.at[pl.ds(HC, HC)], sb).wait()


def _sc_propagate_body(x0_hbm, gidx_hbm, outs_hbm,
                       gi_v, si_v, rbuf0, rbuf1, acc, s0a, s0b, s1a, s1b):
    c = lax.axis_index("c")
    s = lax.axis_index("s")
    row0 = s * ROWS_PER_TILE

    for j in range(ROW_CHUNKS):
        r = row0 + j * CHUNK
        pltpu.sync_copy(x0_hbm.at[pl.ds(r, CHUNK)], rbuf0)
        pltpu.sync_copy(rbuf0, acc.at[pl.ds(r, CHUNK)])
    plsc.subcore_barrier()

    for hop in range(2):
        src_hbm = x0_hbm if hop == 0 else outs_hbm.at[c, 0]

        for stg in range(NCHUNK // GSTAGE):
            pltpu.sync_copy(gidx_hbm.at[c, s, pl.ds(stg * GSTAGE, GSTAGE)], gi_v)
            pltpu.sync_copy(gidx_hbm.at[1 - c, s, pl.ds(stg * GSTAGE, GSTAGE)], si_v)
            _gather2(src_hbm, gi_v, 0, rbuf0, s0a, s0b)

            def edge_pair(m, carry):
                b = 2 * m
                _wait2(src_hbm, gi_v, b, rbuf0, s0a, s0b)
                _gather2(src_hbm, gi_v, b + 1, rbuf1, s1a, s1b)
                pltpu.sync_copy(rbuf0, acc.at[si_v.at[b]], add=True)
                _wait2(src_hbm, gi_v, b + 1, rbuf1, s1a, s1b)

                @pl.when(m < GSTAGE // 2 - 1)
                def _():
                    _gather2(src_hbm, gi_v, b + 2, rbuf0, s0a, s0b)

                pltpu.sync_copy(rbuf1, acc.at[si_v.at[b + 1]], add=True)
                return carry

            lax.fori_loop(0, GSTAGE // 2, edge_pair, 0)
        plsc.subcore_barrier()

        for j in range(ROW_CHUNKS):
            r = row0 + j * CHUNK
            pltpu.sync_copy(acc.at[pl.ds(r, CHUNK)], rbuf0)
            pltpu.sync_copy(rbuf0, outs_hbm.at[c, hop, pl.ds(r, CHUNK)])
        plsc.subcore_barrier()


_sc_propagate = functools.partial(
    pl.kernel,
    out_type=jax.ShapeDtypeStruct((NC, 2, N_PAD, DIM), jnp.float32),
    mesh=plsc.VectorSubcoreMesh(core_axis_name="c", subcore_axis_name="s"),
    scratch_types=[
        pltpu.VMEM((GSTAGE, CHUNK), jnp.int32),
        pltpu.VMEM((GSTAGE, CHUNK), jnp.int32),
        pltpu.VMEM((CHUNK, DIM), jnp.float32),
        pltpu.VMEM((CHUNK, DIM), jnp.float32),
        pltpu.VMEM_SHARED((N_PAD, DIM), jnp.float32),
        pltpu.SemaphoreType.DMA,
        pltpu.SemaphoreType.DMA,
        pltpu.SemaphoreType.DMA,
        pltpu.SemaphoreType.DMA,
    ],
)(_sc_propagate_body)


def _mlp_body(x0_ref, hops_ref, W1_ref, b1_ref, W2_ref, b2_ref, o_ref):
    x0 = x0_ref[...]
    h = jnp.concatenate(
        [x0, hops_ref[0, 0], hops_ref[0, 1], x0, hops_ref[1, 0], hops_ref[1, 1]],
        axis=-1,
    )
    h = jnp.dot(h, W1_ref[...], preferred_element_type=jnp.float32) + b1_ref[...]
    h = jnp.maximum(h, 0.0)
    o_ref[...] = (
        jnp.dot(h, W2_ref[...], preferred_element_type=jnp.float32) + b2_ref[...]
    )


def _mlp(x0p, hops, W1, b1, W2, b2):
    blk = 2000
    grid = N_NODES // blk
    return pl.pallas_call(
        _mlp_body,
        grid=(grid,),
        in_specs=[
            pl.BlockSpec((blk, DIM), lambda i: (i, 0)),
            pl.BlockSpec((NC, 2, blk, DIM), lambda i: (0, 0, i, 0)),
            pl.BlockSpec((2 * 3 * DIM, 512), lambda i: (0, 0)),
            pl.BlockSpec((1, 512), lambda i: (0, 0)),
            pl.BlockSpec((512, DIM), lambda i: (0, 0)),
            pl.BlockSpec((1, DIM), lambda i: (0, 0)),
        ],
        out_specs=pl.BlockSpec((blk, DIM), lambda i: (i, 0)),
        out_shape=jax.ShapeDtypeStruct((N_NODES, DIM), jnp.float32),
    )(x0p, hops, W1, b1.reshape(1, -1), W2, b2.reshape(1, -1))


def kernel(feature, edge_index, W1, b1, W2, b2):
    src = edge_index[0]
    dst = edge_index[1]
    pad = NS * NCHUNK * CHUNK - N_EDGES
    padv = jnp.full((pad,), N_NODES, jnp.int32)
    srcp = jnp.concatenate([src, padv]).reshape(NS, NCHUNK, CHUNK)
    dstp = jnp.concatenate([dst, padv]).reshape(NS, NCHUNK, CHUNK)
    gidx = jnp.stack([srcp, dstp])
    x0p = jnp.concatenate(
        [feature, jnp.zeros((N_PAD - N_NODES, DIM), jnp.float32)]
    )
    hops = _sc_propagate(x0p, gidx)
    return _mlp(x0p, hops, W1, b1, W2, b2)
